# baseline (speedup 1.0000x reference)
import functools

import jax
import jax.numpy as jnp
from jax import lax
from jax.experimental import pallas as pl
from jax.experimental.pallas import tpu as pltpu
from jax.experimental.pallas import tpu_sc as plsc

_ROWS = 8192
_COLS = 1024

_SC_ROWS = 2048
_TC_ROWS = _ROWS - _SC_ROWS

_CHUNK = 128
_NCHUNK = _COLS // _CHUNK
_BR = 1024


def _tc_scan_block(x_ref, o_ref):
    ki = lax.broadcasted_iota(jnp.int32, (_CHUNK, _CHUNK), 0)
    ji = lax.broadcasted_iota(jnp.int32, (_CHUNK, _CHUNK), 1)
    w = jnp.where(ki < ji, 1.0, 0.0).astype(jnp.bfloat16)

    carry = jnp.zeros((_BR, 1), dtype=jnp.float32)
    for c in range(_NCHUNK):
        xc = x_ref[:, c * _CHUNK:(c + 1) * _CHUNK]
        within = lax.dot_general(
            xc.astype(jnp.bfloat16), w, (((1,), (0,)), ((), ())),
            preferred_element_type=jnp.float32,
        )
        out_c = within + carry
        if c == 0:
            lane = lax.broadcasted_iota(jnp.int32, (_BR, _CHUNK), 1)
            out_c = out_c + jnp.where(lane == 0, xc, 0.0)
        o_ref[:, c * _CHUNK:(c + 1) * _CHUNK] = out_c
        carry = carry + jnp.sum(xc, axis=1, keepdims=True)


def _tc_part(x):
    return pl.pallas_call(
        _tc_scan_block,
        grid=(_TC_ROWS // _BR,),
        in_specs=[pl.BlockSpec((_BR, _COLS), lambda i: (i, 0))],
        out_specs=pl.BlockSpec((_BR, _COLS), lambda i: (i, 0)),
        out_shape=jax.ShapeDtypeStruct((_ROWS, _COLS), jnp.float32),
    )(x)


_NC = 2
_NS = 16
_NW = _NC * _NS
_LANES = 16
_GROUPS = 2
_BLK = _GROUPS * _LANES
_ROWS_PER_W = _SC_ROWS // _NW
_NBLK = _ROWS_PER_W // _BLK


def _sc_body(x_hbm, o_hbm, ibuf, obuf):
    wid = lax.axis_index("c") * _NS + lax.axis_index("s")
    row0 = wid * _ROWS_PER_W

    lane = lax.broadcasted_iota(jnp.int32, (_LANES,), 0)
    ridx = [lane + g * _LANES for g in range(_GROUPS)]

    def block_body(b, _):
        base = row0 + b * _BLK
        pltpu.sync_copy(x_hbm.at[pl.ds(base, _BLK), :],
                        ibuf.at[:, pl.ds(0, _COLS)])

        zero = jnp.zeros((_LANES,), jnp.int32)
        accs = []
        for g in range(_GROUPS):
            v0 = plsc.load_gather(ibuf, [ridx[g], zero])
            plsc.store_scatter(obuf, [ridx[g], zero], v0)
            accs.append(v0)

        @plsc.parallel_loop(1, _COLS, step=1, unroll=8, carry=tuple(accs))
        def col_body(j, accs):
            cj = jnp.full((_LANES,), j, jnp.int32)
            out = []
            for g in range(_GROUPS):
                v = plsc.load_gather(ibuf, [ridx[g], cj])
                plsc.store_scatter(obuf, [ridx[g], cj], accs[g])
                out.append(accs[g] + v)
            return tuple(out)

        pltpu.sync_copy(obuf.at[:, pl.ds(0, _COLS)],
                        o_hbm.at[pl.ds(base, _BLK), :])
        return 0

    lax.fori_loop(0, _NBLK, block_body, 0)


def _sc_part(x_sc):
    mesh = plsc.VectorSubcoreMesh(
        core_axis_name="c", subcore_axis_name="s",
        num_cores=_NC, num_subcores=_NS,
    )
    f = functools.partial(
        pl.kernel,
        out_type=jax.ShapeDtypeStruct((_SC_ROWS, _COLS), jnp.float32),
        mesh=mesh,
        scratch_types=[pltpu.VMEM((_BLK, _COLS + 1), jnp.float32),
                       pltpu.VMEM((_BLK, _COLS + 1), jnp.float32)],
        compiler_params=pltpu.CompilerParams(
            use_tc_tiling_on_sc=False, needs_layout_passes=False),
    )(_sc_body)
    return f(x_sc)


def kernel(x):
    y_sc = _sc_part(lax.slice(x, (_TC_ROWS, 0), (_ROWS, _COLS)))
    y_tc = _tc_part(x)
    return lax.dynamic_update_slice(y_tc, y_sc, (_TC_ROWS, 0))


# speedup vs baseline: 3.0702x; 3.0702x over previous
import jax
import jax.numpy as jnp
from jax import lax
from jax.experimental import pallas as pl

_ROWS = 8192
_COLS = 1024
_CHUNK = 128
_NCHUNK = _COLS // _CHUNK
_BR = 2048


def _scan_block(x_ref, o_ref):
    ki = lax.broadcasted_iota(jnp.int32, (_CHUNK, _CHUNK), 0)
    ji = lax.broadcasted_iota(jnp.int32, (_CHUNK, _CHUNK), 1)
    w = jnp.where(ki < ji, 1.0, 0.0).astype(jnp.bfloat16)

    carry = jnp.zeros((_BR, 1), dtype=jnp.float32)
    for c in range(_NCHUNK):
        xc = x_ref[:, c * _CHUNK:(c + 1) * _CHUNK]
        within = lax.dot_general(
            xc.astype(jnp.bfloat16), w, (((1,), (0,)), ((), ())),
            preferred_element_type=jnp.float32,
        )
        out_c = within + carry
        if c == 0:
            lane = lax.broadcasted_iota(jnp.int32, (_BR, _CHUNK), 1)
            out_c = out_c + jnp.where(lane == 0, xc, 0.0)
        o_ref[:, c * _CHUNK:(c + 1) * _CHUNK] = out_c
        carry = carry + jnp.sum(xc, axis=1, keepdims=True)


def kernel(x):
    return pl.pallas_call(
        _scan_block,
        grid=(_ROWS // _BR,),
        in_specs=[pl.BlockSpec((_BR, _COLS), lambda i: (i, 0))],
        out_specs=pl.BlockSpec((_BR, _COLS), lambda i: (i, 0)),
        out_shape=jax.ShapeDtypeStruct((_ROWS, _COLS), jnp.float32),
    )(x)
